# Initial kernel scaffold; baseline (speedup 1.0000x reference)
import jax
import jax.numpy as jnp
from jax.experimental import pallas as pl


def kernel(x, pos, edge_index, batch, params):
    raise NotImplementedError("write your pallas kernel here")


# speedup vs baseline: 1.0398x; 1.0398x over previous
import functools

import jax
import jax.numpy as jnp
from jax import lax
from jax.experimental import pallas as pl
from jax.experimental.pallas import tpu as pltpu


def _pick_block(n, target):
    b = min(n, target)
    while b > 8:
        if n % b == 0 and b % 8 == 0:
            return b
        b -= 8
    return n


def _silu(v):
    return v * jax.nn.sigmoid(v)


def _ln(v, g, b):
    m = jnp.mean(v, axis=-1, keepdims=True)
    va = jnp.mean((v - m) ** 2, axis=-1, keepdims=True)
    return (v - m) / jnp.sqrt(va + 1e-5) * g + b


def _full_spec(a):
    r = len(a.shape)
    return pl.BlockSpec(a.shape, lambda i, *_, _r=r: (0,) * _r)


def _rows_spec(a, blk):
    r = len(a.shape)
    return pl.BlockSpec((blk,) + a.shape[1:], lambda i, *_, _r=r: (i,) + (0,) * (_r - 1))


def _row_kernel(body, n_out_cols, out_dtype, blk, row_args, aux_args):
    n = row_args[0].shape[0]
    grid = (n // blk,)
    in_specs = [_rows_spec(a, blk) for a in row_args] + [_full_spec(a) for a in aux_args]
    return pl.pallas_call(
        body,
        grid=grid,
        in_specs=in_specs,
        out_specs=pl.BlockSpec((blk, n_out_cols), lambda i: (i, 0)),
        out_shape=jax.ShapeDtypeStruct((n, n_out_cols), out_dtype),
    )(*row_args, *aux_args)


def _proj_body(x_ref, w_ref, b_ref, g_ref, bb_ref, o_ref):
    o_ref[...] = _ln(jnp.dot(x_ref[...], w_ref[...]) + b_ref[...], g_ref[...], bb_ref[...])


def _node_body(h_ref, agg_ref, w1h_ref, w1a_ref, b1_ref, w2_ref, b2_ref, g_ref, b_ref, o_ref):
    u = _silu(jnp.dot(h_ref[...], w1h_ref[...]) + jnp.dot(agg_ref[...], w1a_ref[...]) + b1_ref[...])
    o_ref[...] = _ln(h_ref[...] + jnp.dot(u, w2_ref[...]) + b2_ref[...], g_ref[...], b_ref[...])


def _gate_body(h_ref, w1_ref, b1_ref, g_ref, b_ref, w2_ref, b2_ref, w3_ref, b3_ref, o_ref):
    g1 = jax.nn.relu(_ln(jnp.dot(h_ref[...], w1_ref[...]) + b1_ref[...], g_ref[...], b_ref[...]))
    g2 = jax.nn.relu(jnp.dot(g1, w2_ref[...]) + b2_ref[...])
    o_ref[...] = jnp.dot(g2, w3_ref[...]) + b3_ref[...]


def _edge1_body(hr_ref, hc_ref, pr_ref, pc_ref, w1r_ref, w1c_ref, w1p_ref, b1_ref,
                w2_ref, b2_ref, wew_ref, bew_ref, sc_ref, o_ref):
    rel = pr_ref[...] - pc_ref[...]
    pre = (jnp.dot(hr_ref[...], w1r_ref[...]) + jnp.dot(hc_ref[...], w1c_ref[...])
           + jnp.dot(rel, w1p_ref[...]) + b1_ref[...])
    d = jnp.dot(_silu(pre), w2_ref[...]) + b2_ref[...]
    nrm = jnp.maximum(jnp.sqrt(jnp.sum(d * d, axis=-1, keepdims=True)), 1e-8)
    d = d / nrm * sc_ref[...]
    w = jax.nn.sigmoid(jnp.dot(rel, wew_ref[...]) + bew_ref[...])
    o_ref[...] = d * w


def _edge2_body(hr_ref, hc_ref, pr_ref, pc_ref, w1r_ref, w1c_ref, w1p_ref, b1_ref,
                w2_ref, b2_ref, o_ref):
    rel = pr_ref[...] - pc_ref[...]
    pre = (jnp.dot(hr_ref[...], w1r_ref[...]) + jnp.dot(hc_ref[...], w1c_ref[...])
           + jnp.dot(rel, w1p_ref[...]) + b1_ref[...])
    o_ref[...] = _silu(jnp.dot(_silu(pre), w2_ref[...]) + b2_ref[...])


def _pool_body(h_ref, gate_ref, bf_ref, wout_ref, bout_ref, o_ref, m_s, num_s, den_s, *, G):
    p = pl.program_id(0)
    i = pl.program_id(1)
    iota = lax.broadcasted_iota(jnp.int32, (1, G), 1).astype(jnp.float32)
    onehot = bf_ref[...] == iota
    onef = onehot.astype(jnp.float32)

    @pl.when(jnp.logical_and(p == 0, i == 0))
    def _():
        m_s[...] = jnp.full_like(m_s[...], -jnp.inf)

    @pl.when(p == 0)
    def _():
        blk = jnp.max(jnp.where(onehot, gate_ref[...], -jnp.inf), axis=0, keepdims=True)
        m_s[...] = jnp.maximum(m_s[...], blk)

    @pl.when(jnp.logical_and(p == 1, i == 0))
    def _():
        num_s[...] = jnp.zeros_like(num_s[...])
        den_s[...] = jnp.zeros_like(den_s[...])

    @pl.when(p == 1)
    def _():
        m_safe = jnp.where(m_s[...] != -jnp.inf, m_s[...], 0.0)
        mg = jnp.sum(onef * m_safe, axis=1, keepdims=True)
        e = jnp.exp(gate_ref[...] - mg)
        dn = lax.dot_general(onef, e, (((0,), (0,)), ((), ())))
        nm = lax.dot_general(onef, e * h_ref[...], (((0,), (0,)), ((), ())))
        den_s[...] += dn
        num_s[...] += nm

    @pl.when(jnp.logical_and(p == 2, i == 0))
    def _():
        pooled = num_s[...] / (den_s[...] + 1e-16)
        o_ref[...] = jnp.dot(pooled, wout_ref[...]) + bout_ref[...]


def _pool_call(h, gate, batch_f, wout, bout, G):
    n, H = h.shape
    blk = _pick_block(n, 2000)
    grid = (3, n // blk)
    body = functools.partial(_pool_body, G=G)
    return pl.pallas_call(
        body,
        grid=grid,
        in_specs=[
            pl.BlockSpec((blk, H), lambda p, i: (i, 0)),
            pl.BlockSpec((blk, 1), lambda p, i: (i, 0)),
            pl.BlockSpec((blk, 1), lambda p, i: (i, 0)),
            pl.BlockSpec(wout.shape, lambda p, i: (0, 0)),
            pl.BlockSpec(bout.shape, lambda p, i: (0,)),
        ],
        out_specs=pl.BlockSpec((G, wout.shape[1]), lambda p, i: (0, 0)),
        out_shape=jax.ShapeDtypeStruct((G, wout.shape[1]), jnp.float32),
        scratch_shapes=[
            pltpu.VMEM((1, G), jnp.float32),
            pltpu.VMEM((G, H), jnp.float32),
            pltpu.VMEM((G, 1), jnp.float32),
        ],
    )(h, gate, batch_f, wout, bout)


def kernel(x, pos, edge_index, batch, params):
    N, _ = x.shape
    E = edge_index.shape[1]
    H = params["proj"]["w"].shape[1]
    G = 64
    row = edge_index[0]
    col = edge_index[1]

    nblk = _pick_block(N, 2000)
    eblk = _pick_block(E, 4000)

    h = _row_kernel(_proj_body, H, jnp.float32, nblk, [x],
                    [params["proj"]["w"], params["proj"]["b"],
                     params["proj_ln_g"], params["proj_ln_b"]])

    for lp in params["layers"]:
        w1 = lp["coord1"]["w"]
        e1 = lp["edge1"]["w"]
        hr = jnp.take(h, row, axis=0)
        hc = jnp.take(h, col, axis=0)
        pr = jnp.take(pos, row, axis=0)
        pc = jnp.take(pos, col, axis=0)
        d = _row_kernel(_edge1_body, 3, jnp.float32, eblk, [hr, hc, pr, pc],
                        [w1[:H], w1[H:2 * H], w1[2 * H:], lp["coord1"]["b"],
                         lp["coord2"]["w"], lp["coord2"]["b"],
                         lp["ew"]["w"], lp["ew"]["b"], lp["scale"]])
        pos = pos.at[row].add(d)
        pr = jnp.take(pos, row, axis=0)
        pc = jnp.take(pos, col, axis=0)
        msg = _row_kernel(_edge2_body, H, jnp.float32, eblk, [hr, hc, pr, pc],
                          [e1[:H], e1[H:2 * H], e1[2 * H:], lp["edge1"]["b"],
                           lp["edge2"]["w"], lp["edge2"]["b"]])
        agg = jnp.zeros_like(h).at[row].add(msg)
        h = _row_kernel(_node_body, H, jnp.float32, nblk, [h, agg],
                        [lp["node1"]["w"][:H], lp["node1"]["w"][H:], lp["node1"]["b"],
                         lp["node2"]["w"], lp["node2"]["b"], lp["ln_g"], lp["ln_b"]])

    gate = _row_kernel(_gate_body, 1, jnp.float32, nblk, [h],
                       [params["gate1"]["w"], params["gate1"]["b"],
                        params["gate_ln_g"], params["gate_ln_b"],
                        params["gate2"]["w"], params["gate2"]["b"],
                        params["gate3"]["w"], params["gate3"]["b"]])
    batch_f = batch.astype(jnp.float32).reshape(N, 1)
    return _pool_call(h, gate, batch_f, params["out"]["w"], params["out"]["b"], G)


# speedup vs baseline: 1.5129x; 1.4549x over previous
import functools

import jax
import jax.numpy as jnp
from jax import lax
from jax.experimental import pallas as pl
from jax.experimental.pallas import tpu as pltpu
from jax.experimental.pallas import tpu_sc as plsc

_NC, _NS, _NW = 2, 16, 32
_C = 128
_W = 128
_QR = 10112
_AR = 10240


def _pick_block(n, target):
    b = min(n, target)
    while b > 8:
        if n % b == 0 and b % 8 == 0:
            return b
        b -= 8
    return n


def _silu(v):
    return v * jax.nn.sigmoid(v)


def _ln(v, g, b):
    m = jnp.mean(v, axis=-1, keepdims=True)
    va = jnp.mean((v - m) ** 2, axis=-1, keepdims=True)
    return (v - m) / jnp.sqrt(va + 1e-5) * g + b


def _full_spec(a):
    r = len(a.shape)
    return pl.BlockSpec(a.shape, lambda i, *_, _r=r: (0,) * _r)


def _rows_spec(a, blk):
    r = len(a.shape)
    return pl.BlockSpec((blk,) + a.shape[1:], lambda i, *_, _r=r: (i,) + (0,) * (_r - 1))


def _row_kernel(body, n_out_cols, out_dtype, blk, row_args, aux_args):
    n = row_args[0].shape[0]
    grid = (n // blk,)
    in_specs = [_rows_spec(a, blk) for a in row_args] + [_full_spec(a) for a in aux_args]
    return pl.pallas_call(
        body,
        grid=grid,
        in_specs=in_specs,
        out_specs=pl.BlockSpec((blk, n_out_cols), lambda i: (i, 0)),
        out_shape=jax.ShapeDtypeStruct((n, n_out_cols), out_dtype),
    )(*row_args, *aux_args)


def _mesh():
    return plsc.VectorSubcoreMesh(core_axis_name="c", subcore_axis_name="s")


def _strided_loop(n_items, lane, n_lanes, step_fn):
    full, extra = divmod(n_items, n_lanes)
    nj = jnp.where(lane < extra, full + 1, full)

    def body(j, carry):
        step_fn(lane + j * n_lanes)
        return carry

    lax.fori_loop(0, nj, body, 0)


def _range_copy(src, dst, sbase, dbase, nrows, bounce):
    fc, tl = divmod(nrows, _C)
    for k in range(fc):
        pltpu.sync_copy(src.at[pl.ds(sbase + k * _C, _C)], bounce)
        pltpu.sync_copy(bounce, dst.at[pl.ds(dbase + k * _C, _C)])
    if tl:
        pltpu.sync_copy(src.at[pl.ds(sbase + fc * _C, tl)], bounce.at[pl.ds(0, tl)])
        pltpu.sync_copy(bounce.at[pl.ds(0, tl)], dst.at[pl.ds(dbase + fc * _C, tl)])


def _gatherT_call(T, row, col):
    E = row.shape[0]
    n_chunks = E // _C
    out_type = [jax.ShapeDtypeStruct((E, _W), jnp.float32),
                jax.ShapeDtypeStruct((E, _W), jnp.float32)]
    scratch = [pltpu.VMEM((_C,), jnp.int32), pltpu.VMEM((_C,), jnp.int32),
               pltpu.VMEM((_C, _W), jnp.float32), pltpu.VMEM((_C, _W), jnp.float32),
               pltpu.SemaphoreType.DMA]

    @functools.partial(pl.kernel, mesh=_mesh(), out_type=out_type, scratch_types=scratch)
    def run(t_h, row_h, col_h, r_o, c_o, iv, iv2, bufr, bufc, sem):
        w = lax.axis_index("s") * _NC + lax.axis_index("c")

        def step(cid):
            o = pl.multiple_of(cid * _C, _C)
            pltpu.sync_copy(row_h.at[pl.ds(o, _C)], iv)
            pltpu.sync_copy(col_h.at[pl.ds(o, _C)], iv2)
            c1 = pltpu.async_copy(t_h.at[iv], bufr, sem)
            c2 = pltpu.async_copy(t_h.at[iv2], bufc, sem)
            c1.wait()
            c2.wait()
            pltpu.sync_copy(bufr, r_o.at[pl.ds(o, _C)])
            pltpu.sync_copy(bufc, c_o.at[pl.ds(o, _C)])

        _strided_loop(n_chunks, w, _NW, step)

    return run(T, row, col)


def _scatter128_call(init_tab, data, idx, zero_init):
    E = data.shape[0]
    N = init_tab.shape[0]
    n_chunks = E // _C
    quarters = [(k * _QR, min(_QR, N - k * _QR)) for k in range((N + _QR - 1) // _QR)]
    zb = jnp.zeros((_C, _W), jnp.float32)
    out_type = [jax.ShapeDtypeStruct((N, _W), jnp.float32)]
    scratch = [pltpu.VMEM((_C,), jnp.int32), pltpu.VMEM((_C,), jnp.int32),
               pltpu.VMEM((_C, _W), jnp.float32), pltpu.VMEM((_C, _W), jnp.float32),
               pltpu.VMEM_SHARED((_AR, _W), jnp.float32)]

    @functools.partial(pl.kernel, mesh=_mesh(), out_type=out_type, scratch_types=scratch)
    def run(t_h, d_h, idx_h, out_h, iv, ivl, buf, zbuf, acc):
        c = lax.axis_index("c")
        s = lax.axis_index("s")
        if zero_init:
            pltpu.sync_copy(t_h, zbuf)

        def quarter(qstart, qsize):
            pb = (qsize // _NS) // 8 * 8
            ptail = qsize - (_NS - 1) * pb

            @pl.when(s < _NS - 1)
            def _():
                if zero_init:
                    fc, tl = divmod(pb, _C)
                    for k in range(fc):
                        pltpu.sync_copy(zbuf, acc.at[pl.ds(s * pb + k * _C, _C)])
                    if tl:
                        pltpu.sync_copy(zbuf.at[pl.ds(0, tl)],
                                        acc.at[pl.ds(s * pb + fc * _C, tl)])
                else:
                    _range_copy(t_h, acc, qstart + s * pb, s * pb, pb, buf)

            @pl.when(s == _NS - 1)
            def _():
                base = (_NS - 1) * pb
                if zero_init:
                    fc, tl = divmod(ptail, _C)
                    for k in range(fc):
                        pltpu.sync_copy(zbuf, acc.at[pl.ds(base + k * _C, _C)])
                    if tl:
                        pltpu.sync_copy(zbuf.at[pl.ds(0, tl)],
                                        acc.at[pl.ds(base + fc * _C, tl)])
                else:
                    _range_copy(t_h, acc, qstart + base, base, ptail, buf)
            plsc.subcore_barrier()

            def st(cid):
                o = pl.multiple_of(cid * _C, _C)
                pltpu.sync_copy(idx_h.at[pl.ds(o, _C)], iv)
                for k in range(_C // 16):
                    v = iv[pl.ds(k * 16, 16)] - qstart
                    oob = jnp.logical_or(v < 0, v >= qsize)
                    ivl[pl.ds(k * 16, 16)] = jnp.where(oob, _QR, v)
                pltpu.sync_copy(d_h.at[pl.ds(o, _C)], buf)
                pltpu.sync_copy(buf, acc.at[ivl], add=True)

            _strided_loop(n_chunks, s, _NS, st)
            plsc.subcore_barrier()

            @pl.when(s < _NS - 1)
            def _():
                _range_copy(acc, out_h, s * pb, qstart + s * pb, pb, buf)

            @pl.when(s == _NS - 1)
            def _():
                base = (_NS - 1) * pb
                _range_copy(acc, out_h, base, qstart + base, ptail, buf)

        @pl.when(c == 0)
        def _():
            for qi, q in enumerate(quarters[0::2]):
                if qi:
                    plsc.subcore_barrier()
                quarter(*q)

        @pl.when(c == 1)
        def _():
            for qi, q in enumerate(quarters[1::2]):
                if qi:
                    plsc.subcore_barrier()
                quarter(*q)

    return run(zb if zero_init else init_tab, data, idx)[0]


def _proj_body(x_ref, p_ref, w_ref, b_ref, g_ref, bb_ref, o_ref):
    h = _ln(jnp.dot(x_ref[...], w_ref[...]) + b_ref[...], g_ref[...], bb_ref[...])
    z = jnp.zeros((h.shape[0], 61), jnp.float32)
    o_ref[...] = jnp.concatenate([h, p_ref[...], z], axis=1)


def _edge1_body(r_ref, c_ref, w1r_ref, w1c_ref, w1p_ref, b1_ref,
                w2_ref, b2_ref, wew_ref, bew_ref, sc_ref, o_ref):
    hr = r_ref[...][:, :64]
    hc = c_ref[...][:, :64]
    rel = r_ref[...][:, 64:67] - c_ref[...][:, 64:67]
    pre = (jnp.dot(hr, w1r_ref[...]) + jnp.dot(hc, w1c_ref[...])
           + jnp.dot(rel, w1p_ref[...]) + b1_ref[...])
    d = jnp.dot(_silu(pre), w2_ref[...]) + b2_ref[...]
    nrm = jnp.maximum(jnp.sqrt(jnp.sum(d * d, axis=-1, keepdims=True)), 1e-8)
    d = d / nrm * sc_ref[...]
    w = jax.nn.sigmoid(jnp.dot(rel, wew_ref[...]) + bew_ref[...])
    dw = d * w
    o_ref[...] = jnp.concatenate([dw, jnp.zeros((dw.shape[0], 125), jnp.float32)], axis=1)


def _edge2_body(r_ref, c_ref, pr_ref, pc_ref, w1r_ref, w1c_ref, w1p_ref, b1_ref,
                w2_ref, b2_ref, o_ref):
    hr = r_ref[...][:, :64]
    hc = c_ref[...][:, :64]
    rel = pr_ref[...][:, :3] - pc_ref[...][:, :3]
    pre = (jnp.dot(hr, w1r_ref[...]) + jnp.dot(hc, w1c_ref[...])
           + jnp.dot(rel, w1p_ref[...]) + b1_ref[...])
    msg = _silu(jnp.dot(_silu(pre), w2_ref[...]) + b2_ref[...])
    o_ref[...] = jnp.concatenate([msg, jnp.zeros((msg.shape[0], 64), jnp.float32)], axis=1)


def _node_body(t_ref, agg_ref, pn_ref, w1h_ref, w1a_ref, b1_ref,
               w2_ref, b2_ref, g_ref, b_ref, o_ref):
    h = t_ref[...][:, :64]
    agg = agg_ref[...][:, :64]
    u = _silu(jnp.dot(h, w1h_ref[...]) + jnp.dot(agg, w1a_ref[...]) + b1_ref[...])
    hn = _ln(h + jnp.dot(u, w2_ref[...]) + b2_ref[...], g_ref[...], b_ref[...])
    z = jnp.zeros((hn.shape[0], 61), jnp.float32)
    o_ref[...] = jnp.concatenate([hn, pn_ref[...][:, :3], z], axis=1)


def _gate_body(t_ref, w1_ref, b1_ref, g_ref, b_ref, w2_ref, b2_ref, w3_ref, b3_ref, o_ref):
    h = t_ref[...][:, :64]
    g1 = jax.nn.relu(_ln(jnp.dot(h, w1_ref[...]) + b1_ref[...], g_ref[...], b_ref[...]))
    g2 = jax.nn.relu(jnp.dot(g1, w2_ref[...]) + b2_ref[...])
    o_ref[...] = jnp.dot(g2, w3_ref[...]) + b3_ref[...]


def _pool_body(t_ref, gate_ref, bf_ref, wout_ref, bout_ref, o_ref, m_s, num_s, den_s, *, G):
    p = pl.program_id(0)
    i = pl.program_id(1)
    iota = lax.broadcasted_iota(jnp.int32, (1, G), 1).astype(jnp.float32)
    onehot = bf_ref[...] == iota
    onef = onehot.astype(jnp.float32)

    @pl.when(jnp.logical_and(p == 0, i == 0))
    def _():
        m_s[...] = jnp.full_like(m_s[...], -jnp.inf)

    @pl.when(p == 0)
    def _():
        blk = jnp.max(jnp.where(onehot, gate_ref[...], -jnp.inf), axis=0, keepdims=True)
        m_s[...] = jnp.maximum(m_s[...], blk)

    @pl.when(jnp.logical_and(p == 1, i == 0))
    def _():
        num_s[...] = jnp.zeros_like(num_s[...])
        den_s[...] = jnp.zeros_like(den_s[...])

    @pl.when(p == 1)
    def _():
        h = t_ref[...][:, :64]
        m_safe = jnp.where(m_s[...] != -jnp.inf, m_s[...], 0.0)
        mg = jnp.sum(onef * m_safe, axis=1, keepdims=True)
        e = jnp.exp(gate_ref[...] - mg)
        den_s[...] += lax.dot_general(onef, e, (((0,), (0,)), ((), ())))
        num_s[...] += lax.dot_general(onef, e * h, (((0,), (0,)), ((), ())))

    @pl.when(jnp.logical_and(p == 2, i == 0))
    def _():
        pooled = num_s[...] / (den_s[...] + 1e-16)
        o_ref[...] = jnp.dot(pooled, wout_ref[...]) + bout_ref[...]


def _pool_call(T, gate, batch_f, wout, bout, G):
    n = T.shape[0]
    H = 64
    blk = _pick_block(n, 2000)
    grid = (3, n // blk)
    body = functools.partial(_pool_body, G=G)
    return pl.pallas_call(
        body,
        grid=grid,
        in_specs=[
            pl.BlockSpec((blk, T.shape[1]), lambda p, i: (i, 0)),
            pl.BlockSpec((blk, 1), lambda p, i: (i, 0)),
            pl.BlockSpec((blk, 1), lambda p, i: (i, 0)),
            pl.BlockSpec(wout.shape, lambda p, i: (0, 0)),
            pl.BlockSpec(bout.shape, lambda p, i: (0,)),
        ],
        out_specs=pl.BlockSpec((G, wout.shape[1]), lambda p, i: (0, 0)),
        out_shape=jax.ShapeDtypeStruct((G, wout.shape[1]), jnp.float32),
        scratch_shapes=[
            pltpu.VMEM((1, G), jnp.float32),
            pltpu.VMEM((G, H), jnp.float32),
            pltpu.VMEM((G, 1), jnp.float32),
        ],
    )(T, gate, batch_f, wout, bout)


def kernel(x, pos, edge_index, batch, params):
    N, _ = x.shape
    E = edge_index.shape[1]
    H = params["proj"]["w"].shape[1]
    G = 64
    row = edge_index[0]
    col = edge_index[1]

    nblk = _pick_block(N, 2000)
    eblk = _pick_block(E, 4000)

    posT = jnp.pad(pos, ((0, 0), (0, _W - 3)))

    T = _row_kernel(_proj_body, _W, jnp.float32, nblk, [x, pos],
                    [params["proj"]["w"], params["proj"]["b"],
                     params["proj_ln_g"], params["proj_ln_b"]])

    for lp in params["layers"]:
        w1 = lp["coord1"]["w"]
        e1 = lp["edge1"]["w"]
        R, Cc = _gatherT_call(T, row, col)
        d = _row_kernel(_edge1_body, _W, jnp.float32, eblk, [R, Cc],
                        [w1[:H], w1[H:2 * H], w1[2 * H:], lp["coord1"]["b"],
                         lp["coord2"]["w"], lp["coord2"]["b"],
                         lp["ew"]["w"], lp["ew"]["b"], lp["scale"]])
        posT = _scatter128_call(posT, d, row, zero_init=False)
        pr2, pc2 = _gatherT_call(posT, row, col)
        msg = _row_kernel(_edge2_body, _W, jnp.float32, eblk, [R, Cc, pr2, pc2],
                          [e1[:H], e1[H:2 * H], e1[2 * H:], lp["edge1"]["b"],
                           lp["edge2"]["w"], lp["edge2"]["b"]])
        agg = _scatter128_call(posT, msg, row, zero_init=True)
        T = _row_kernel(_node_body, _W, jnp.float32, nblk, [T, agg, posT],
                        [lp["node1"]["w"][:H], lp["node1"]["w"][H:], lp["node1"]["b"],
                         lp["node2"]["w"], lp["node2"]["b"], lp["ln_g"], lp["ln_b"]])

    gate = _row_kernel(_gate_body, 1, jnp.float32, nblk, [T],
                       [params["gate1"]["w"], params["gate1"]["b"],
                        params["gate_ln_g"], params["gate_ln_b"],
                        params["gate2"]["w"], params["gate2"]["b"],
                        params["gate3"]["w"], params["gate3"]["b"]])
    batch_f = batch.astype(jnp.float32).reshape(N, 1)
    return _pool_call(T, gate, batch_f, params["out"]["w"], params["out"]["b"], G)


# speedup vs baseline: 1.6064x; 1.0618x over previous
import functools

import jax
import jax.numpy as jnp
from jax import lax
from jax.experimental import pallas as pl
from jax.experimental.pallas import tpu as pltpu
from jax.experimental.pallas import tpu_sc as plsc

_NC, _NS, _NW = 2, 16, 32
_C = 128
_W = 128
_QR = 10112
_AR = 10240


def _pick_block(n, target):
    b = min(n, target)
    while b > 8:
        if n % b == 0 and b % 8 == 0:
            return b
        b -= 8
    return n


def _silu(v):
    return v * jax.nn.sigmoid(v)


def _ln(v, g, b):
    m = jnp.mean(v, axis=-1, keepdims=True)
    va = jnp.mean((v - m) ** 2, axis=-1, keepdims=True)
    return (v - m) / jnp.sqrt(va + 1e-5) * g + b


def _full_spec(a):
    r = len(a.shape)
    return pl.BlockSpec(a.shape, lambda i, *_, _r=r: (0,) * _r)


def _rows_spec(a, blk):
    r = len(a.shape)
    return pl.BlockSpec((blk,) + a.shape[1:], lambda i, *_, _r=r: (i,) + (0,) * (_r - 1))


def _row_kernel(body, n_out_cols, out_dtype, blk, row_args, aux_args):
    n = row_args[0].shape[0]
    grid = (n // blk,)
    in_specs = [_rows_spec(a, blk) for a in row_args] + [_full_spec(a) for a in aux_args]
    return pl.pallas_call(
        body,
        grid=grid,
        in_specs=in_specs,
        out_specs=pl.BlockSpec((blk, n_out_cols), lambda i: (i, 0)),
        out_shape=jax.ShapeDtypeStruct((n, n_out_cols), out_dtype),
    )(*row_args, *aux_args)


def _mesh():
    return plsc.VectorSubcoreMesh(core_axis_name="c", subcore_axis_name="s")


def _strided_loop(n_items, lane, n_lanes, step_fn):
    full, extra = divmod(n_items, n_lanes)
    nj = jnp.where(lane < extra, full + 1, full)

    def body(j, carry):
        step_fn(lane + j * n_lanes)
        return carry

    lax.fori_loop(0, nj, body, 0)


def _range_copy(src, dst, sbase, dbase, nrows, bounce):
    fc, tl = divmod(nrows, _C)
    for k in range(fc):
        pltpu.sync_copy(src.at[pl.ds(sbase + k * _C, _C)], bounce)
        pltpu.sync_copy(bounce, dst.at[pl.ds(dbase + k * _C, _C)])
    if tl:
        pltpu.sync_copy(src.at[pl.ds(sbase + fc * _C, tl)], bounce.at[pl.ds(0, tl)])
        pltpu.sync_copy(bounce.at[pl.ds(0, tl)], dst.at[pl.ds(dbase + fc * _C, tl)])


def _gatherT_call(T, row, col):
    E = row.shape[0]
    n_chunks = E // _C
    nj = -(-n_chunks // _NW)
    if nj % 2:
        nj += 1
    out_type = [jax.ShapeDtypeStruct((E, _W), jnp.float32),
                jax.ShapeDtypeStruct((E, _W), jnp.float32)]
    scratch = [pltpu.VMEM((2, _C), jnp.int32), pltpu.VMEM((2, _C), jnp.int32),
               pltpu.VMEM((2, _C, _W), jnp.float32), pltpu.VMEM((2, _C, _W), jnp.float32),
               pltpu.SemaphoreType.DMA, pltpu.SemaphoreType.DMA]

    @functools.partial(pl.kernel, mesh=_mesh(), out_type=out_type, scratch_types=scratch)
    def run(t_h, row_h, col_h, r_o, c_o, iv, iv2, bufr, bufc, sem0, sem1):
        w = lax.axis_index("s") * _NC + lax.axis_index("c")
        sems = [sem0, sem1]

        def body(j2, carry):
            cids = []
            for b in range(2):
                cid = jnp.minimum(w + (j2 * 2 + b) * _NW, n_chunks - 1)
                o = pl.multiple_of(cid * _C, _C)
                cids.append(o)
                pltpu.sync_copy(row_h.at[pl.ds(o, _C)], iv.at[b])
                pltpu.sync_copy(col_h.at[pl.ds(o, _C)], iv2.at[b])
                pltpu.async_copy(t_h.at[iv.at[b]], bufr.at[b], sems[b])
                pltpu.async_copy(t_h.at[iv2.at[b]], bufc.at[b], sems[b])
            for b in range(2):
                o = cids[b]
                pltpu.make_async_copy(t_h.at[iv.at[b]], bufr.at[b], sems[b]).wait()
                pltpu.make_async_copy(t_h.at[iv2.at[b]], bufc.at[b], sems[b]).wait()
                pltpu.sync_copy(bufr.at[b], r_o.at[pl.ds(o, _C)])
                pltpu.sync_copy(bufc.at[b], c_o.at[pl.ds(o, _C)])
            return carry

        lax.fori_loop(0, nj // 2, body, 0)

    return run(T, row, col)


def _scatter128_call(init_tab, data, idx, zero_init):
    E = data.shape[0]
    N = init_tab.shape[0]
    n_chunks = E // _C
    quarters = [(k * _QR, min(_QR, N - k * _QR)) for k in range((N + _QR - 1) // _QR)]
    zb = jnp.zeros((_C, _W), jnp.float32)
    out_type = [jax.ShapeDtypeStruct((N, _W), jnp.float32)]
    scratch = [pltpu.VMEM((_C,), jnp.int32), pltpu.VMEM((_C,), jnp.int32),
               pltpu.VMEM((_C, _W), jnp.float32), pltpu.VMEM((_C, _W), jnp.float32),
               pltpu.VMEM_SHARED((_AR, _W), jnp.float32)]

    @functools.partial(pl.kernel, mesh=_mesh(), out_type=out_type, scratch_types=scratch)
    def run(t_h, d_h, idx_h, out_h, iv, ivl, buf, zbuf, acc):
        c = lax.axis_index("c")
        s = lax.axis_index("s")
        if zero_init:
            pltpu.sync_copy(t_h, zbuf)

        def quarter(qstart, qsize):
            pb = (qsize // _NS) // 8 * 8
            ptail = qsize - (_NS - 1) * pb

            @pl.when(s < _NS - 1)
            def _():
                if zero_init:
                    fc, tl = divmod(pb, _C)
                    for k in range(fc):
                        pltpu.sync_copy(zbuf, acc.at[pl.ds(s * pb + k * _C, _C)])
                    if tl:
                        pltpu.sync_copy(zbuf.at[pl.ds(0, tl)],
                                        acc.at[pl.ds(s * pb + fc * _C, tl)])
                else:
                    _range_copy(t_h, acc, qstart + s * pb, s * pb, pb, buf)

            @pl.when(s == _NS - 1)
            def _():
                base = (_NS - 1) * pb
                if zero_init:
                    fc, tl = divmod(ptail, _C)
                    for k in range(fc):
                        pltpu.sync_copy(zbuf, acc.at[pl.ds(base + k * _C, _C)])
                    if tl:
                        pltpu.sync_copy(zbuf.at[pl.ds(0, tl)],
                                        acc.at[pl.ds(base + fc * _C, tl)])
                else:
                    _range_copy(t_h, acc, qstart + base, base, ptail, buf)
            plsc.subcore_barrier()

            def st(cid):
                o = pl.multiple_of(cid * _C, _C)
                pltpu.sync_copy(idx_h.at[pl.ds(o, _C)], iv)
                for k in range(_C // 16):
                    v = iv[pl.ds(k * 16, 16)] - qstart
                    oob = jnp.logical_or(v < 0, v >= qsize)
                    ivl[pl.ds(k * 16, 16)] = jnp.where(oob, _QR, v)
                pltpu.sync_copy(d_h.at[pl.ds(o, _C)], buf)
                pltpu.sync_copy(buf, acc.at[ivl], add=True)

            _strided_loop(n_chunks, s, _NS, st)
            plsc.subcore_barrier()

            @pl.when(s < _NS - 1)
            def _():
                _range_copy(acc, out_h, s * pb, qstart + s * pb, pb, buf)

            @pl.when(s == _NS - 1)
            def _():
                base = (_NS - 1) * pb
                _range_copy(acc, out_h, base, qstart + base, ptail, buf)

        @pl.when(c == 0)
        def _():
            for qi, q in enumerate(quarters[0::2]):
                if qi:
                    plsc.subcore_barrier()
                quarter(*q)

        @pl.when(c == 1)
        def _():
            for qi, q in enumerate(quarters[1::2]):
                if qi:
                    plsc.subcore_barrier()
                quarter(*q)

    return run(zb if zero_init else init_tab, data, idx)[0]


def _proj_body(x_ref, p_ref, w_ref, b_ref, g_ref, bb_ref, o_ref):
    h = _ln(jnp.dot(x_ref[...], w_ref[...]) + b_ref[...], g_ref[...], bb_ref[...])
    z = jnp.zeros((h.shape[0], 61), jnp.float32)
    o_ref[...] = jnp.concatenate([h, p_ref[...], z], axis=1)


def _edge1_body(r_ref, c_ref, w1r_ref, w1c_ref, w1p_ref, b1_ref,
                w2_ref, b2_ref, wew_ref, bew_ref, sc_ref, o_ref):
    hr = r_ref[...][:, :64]
    hc = c_ref[...][:, :64]
    rel = r_ref[...][:, 64:67] - c_ref[...][:, 64:67]
    pre = (jnp.dot(hr, w1r_ref[...]) + jnp.dot(hc, w1c_ref[...])
           + jnp.dot(rel, w1p_ref[...]) + b1_ref[...])
    d = jnp.dot(_silu(pre), w2_ref[...]) + b2_ref[...]
    nrm = jnp.maximum(jnp.sqrt(jnp.sum(d * d, axis=-1, keepdims=True)), 1e-8)
    d = d / nrm * sc_ref[...]
    w = jax.nn.sigmoid(jnp.dot(rel, wew_ref[...]) + bew_ref[...])
    dw = d * w
    o_ref[...] = jnp.concatenate([dw, jnp.zeros((dw.shape[0], 125), jnp.float32)], axis=1)


def _edge2_body(r_ref, c_ref, pr_ref, pc_ref, w1r_ref, w1c_ref, w1p_ref, b1_ref,
                w2_ref, b2_ref, o_ref):
    hr = r_ref[...][:, :64]
    hc = c_ref[...][:, :64]
    rel = pr_ref[...][:, :3] - pc_ref[...][:, :3]
    pre = (jnp.dot(hr, w1r_ref[...]) + jnp.dot(hc, w1c_ref[...])
           + jnp.dot(rel, w1p_ref[...]) + b1_ref[...])
    msg = _silu(jnp.dot(_silu(pre), w2_ref[...]) + b2_ref[...])
    o_ref[...] = jnp.concatenate([msg, jnp.zeros((msg.shape[0], 64), jnp.float32)], axis=1)


def _node_body(t_ref, agg_ref, pn_ref, w1h_ref, w1a_ref, b1_ref,
               w2_ref, b2_ref, g_ref, b_ref, o_ref):
    h = t_ref[...][:, :64]
    agg = agg_ref[...][:, :64]
    u = _silu(jnp.dot(h, w1h_ref[...]) + jnp.dot(agg, w1a_ref[...]) + b1_ref[...])
    hn = _ln(h + jnp.dot(u, w2_ref[...]) + b2_ref[...], g_ref[...], b_ref[...])
    z = jnp.zeros((hn.shape[0], 61), jnp.float32)
    o_ref[...] = jnp.concatenate([hn, pn_ref[...][:, :3], z], axis=1)


def _gate_body(t_ref, w1_ref, b1_ref, g_ref, b_ref, w2_ref, b2_ref, w3_ref, b3_ref, o_ref):
    h = t_ref[...][:, :64]
    g1 = jax.nn.relu(_ln(jnp.dot(h, w1_ref[...]) + b1_ref[...], g_ref[...], b_ref[...]))
    g2 = jax.nn.relu(jnp.dot(g1, w2_ref[...]) + b2_ref[...])
    o_ref[...] = jnp.dot(g2, w3_ref[...]) + b3_ref[...]


def _pool_body(t_ref, gate_ref, bf_ref, wout_ref, bout_ref, o_ref, m_s, num_s, den_s, *, G):
    p = pl.program_id(0)
    i = pl.program_id(1)
    iota = lax.broadcasted_iota(jnp.int32, (1, G), 1).astype(jnp.float32)
    onehot = bf_ref[...] == iota
    onef = onehot.astype(jnp.float32)

    @pl.when(jnp.logical_and(p == 0, i == 0))
    def _():
        m_s[...] = jnp.full_like(m_s[...], -jnp.inf)

    @pl.when(p == 0)
    def _():
        blk = jnp.max(jnp.where(onehot, gate_ref[...], -jnp.inf), axis=0, keepdims=True)
        m_s[...] = jnp.maximum(m_s[...], blk)

    @pl.when(jnp.logical_and(p == 1, i == 0))
    def _():
        num_s[...] = jnp.zeros_like(num_s[...])
        den_s[...] = jnp.zeros_like(den_s[...])

    @pl.when(p == 1)
    def _():
        h = t_ref[...][:, :64]
        m_safe = jnp.where(m_s[...] != -jnp.inf, m_s[...], 0.0)
        mg = jnp.sum(onef * m_safe, axis=1, keepdims=True)
        e = jnp.exp(gate_ref[...] - mg)
        den_s[...] += lax.dot_general(onef, e, (((0,), (0,)), ((), ())))
        num_s[...] += lax.dot_general(onef, e * h, (((0,), (0,)), ((), ())))

    @pl.when(jnp.logical_and(p == 2, i == 0))
    def _():
        pooled = num_s[...] / (den_s[...] + 1e-16)
        o_ref[...] = jnp.dot(pooled, wout_ref[...]) + bout_ref[...]


def _pool_call(T, gate, batch_f, wout, bout, G):
    n = T.shape[0]
    H = 64
    blk = _pick_block(n, 2000)
    grid = (3, n // blk)
    body = functools.partial(_pool_body, G=G)
    return pl.pallas_call(
        body,
        grid=grid,
        in_specs=[
            pl.BlockSpec((blk, T.shape[1]), lambda p, i: (i, 0)),
            pl.BlockSpec((blk, 1), lambda p, i: (i, 0)),
            pl.BlockSpec((blk, 1), lambda p, i: (i, 0)),
            pl.BlockSpec(wout.shape, lambda p, i: (0, 0)),
            pl.BlockSpec(bout.shape, lambda p, i: (0,)),
        ],
        out_specs=pl.BlockSpec((G, wout.shape[1]), lambda p, i: (0, 0)),
        out_shape=jax.ShapeDtypeStruct((G, wout.shape[1]), jnp.float32),
        scratch_shapes=[
            pltpu.VMEM((1, G), jnp.float32),
            pltpu.VMEM((G, H), jnp.float32),
            pltpu.VMEM((G, 1), jnp.float32),
        ],
    )(T, gate, batch_f, wout, bout)


def kernel(x, pos, edge_index, batch, params):
    N, _ = x.shape
    E = edge_index.shape[1]
    H = params["proj"]["w"].shape[1]
    G = 64
    row = edge_index[0]
    col = edge_index[1]

    nblk = _pick_block(N, 2000)
    eblk = _pick_block(E, 4000)

    posT = jnp.pad(pos, ((0, 0), (0, _W - 3)))

    T = _row_kernel(_proj_body, _W, jnp.float32, nblk, [x, pos],
                    [params["proj"]["w"], params["proj"]["b"],
                     params["proj_ln_g"], params["proj_ln_b"]])

    for lp in params["layers"]:
        w1 = lp["coord1"]["w"]
        e1 = lp["edge1"]["w"]
        R, Cc = _gatherT_call(T, row, col)
        d = _row_kernel(_edge1_body, _W, jnp.float32, eblk, [R, Cc],
                        [w1[:H], w1[H:2 * H], w1[2 * H:], lp["coord1"]["b"],
                         lp["coord2"]["w"], lp["coord2"]["b"],
                         lp["ew"]["w"], lp["ew"]["b"], lp["scale"]])
        posT = _scatter128_call(posT, d, row, zero_init=False)
        pr2, pc2 = _gatherT_call(posT, row, col)
        msg = _row_kernel(_edge2_body, _W, jnp.float32, eblk, [R, Cc, pr2, pc2],
                          [e1[:H], e1[H:2 * H], e1[2 * H:], lp["edge1"]["b"],
                           lp["edge2"]["w"], lp["edge2"]["b"]])
        agg = _scatter128_call(posT, msg, row, zero_init=True)
        T = _row_kernel(_node_body, _W, jnp.float32, nblk, [T, agg, posT],
                        [lp["node1"]["w"][:H], lp["node1"]["w"][H:], lp["node1"]["b"],
                         lp["node2"]["w"], lp["node2"]["b"], lp["ln_g"], lp["ln_b"]])

    gate = _row_kernel(_gate_body, 1, jnp.float32, nblk, [T],
                       [params["gate1"]["w"], params["gate1"]["b"],
                        params["gate_ln_g"], params["gate_ln_b"],
                        params["gate2"]["w"], params["gate2"]["b"],
                        params["gate3"]["w"], params["gate3"]["b"]])
    batch_f = batch.astype(jnp.float32).reshape(N, 1)
    return _pool_call(T, gate, batch_f, params["out"]["w"], params["out"]["b"], G)
